# Initial kernel scaffold; baseline (speedup 1.0000x reference)
import jax
import jax.numpy as jnp
from jax.experimental import pallas as pl


def kernel(x, table, W, b):
    raise NotImplementedError("write your pallas kernel here")


# speedup vs baseline: 2.1893x; 2.1893x over previous
import functools

import jax
import jax.numpy as jnp
from jax import lax
from jax.experimental import pallas as pl
from jax.experimental.pallas import tpu as pltpu
from jax.experimental.pallas import tpu_sc as plsc

VOCAB = 1000000
EMBED = 32
NCLS = 2
BATCH = 4096
SEQ = 200

NC = 2
NS = 16
L = 16
NW = NC * NS
BPW = BATCH // NW
CH = 4
HALF = SEQ // 2
NCHUNK = BPW // CH
UNROLL = 4

_mesh = plsc.VectorSubcoreMesh(core_axis_name="c", subcore_axis_name="s")


@functools.partial(
    pl.kernel,
    mesh=_mesh,
    compiler_params=pltpu.CompilerParams(use_tc_tiling_on_sc=False),
    out_type=jax.ShapeDtypeStruct((BATCH, EMBED), jnp.float32),
    scratch_types=[
        pltpu.VMEM((2 * CH, HALF), jnp.int32),
        pltpu.VMEM((2 * CH, HALF, EMBED), jnp.float32),
        pltpu.VMEM((CH, EMBED), jnp.float32),
        pltpu.SemaphoreType.DMA,
    ],
)
def _pooled_sum(x_hbm, table_hbm, out_hbm, idx_v, rows_v, acc_v, sem):
    wid = lax.axis_index("s") * NC + lax.axis_index("c")
    base = wid * BPW

    def chunk_body(ci, carry):
        row0 = base + ci * CH
        pltpu.sync_copy(x_hbm.at[pl.ds(row0 * 2, 2 * CH)], idx_v)
        cps = [
            pltpu.async_copy(table_hbm.at[idx_v.at[j]], rows_v.at[j], sem)
            for j in range(2 * CH)
        ]
        for cp in cps:
            cp.wait()
        for r in range(CH):
            def sbody(i, acc, _r=r):
                a0, a1 = acc
                for k in range(UNROLL):
                    s = i * UNROLL + k
                    a0 = a0 + rows_v[2 * _r, s, pl.ds(0, L)]
                    a1 = a1 + rows_v[2 * _r, s, pl.ds(L, L)]
                    a0 = a0 + rows_v[2 * _r + 1, s, pl.ds(0, L)]
                    a1 = a1 + rows_v[2 * _r + 1, s, pl.ds(L, L)]
                return a0, a1
            z = jnp.zeros((L,), jnp.float32)
            a0, a1 = lax.fori_loop(0, HALF // UNROLL, sbody, (z, z))
            acc_v[r, pl.ds(0, L)] = a0
            acc_v[r, pl.ds(L, L)] = a1
        pltpu.sync_copy(acc_v, out_hbm.at[pl.ds(row0, CH)])
        return carry

    lax.fori_loop(0, NCHUNK, chunk_body, 0)


def _head_body(p_ref, w_ref, b_ref, o_ref):
    pooled = p_ref[...] * (1.0 / SEQ)
    o_ref[...] = (
        jnp.dot(pooled, w_ref[...], preferred_element_type=jnp.float32)
        + b_ref[...]
    )


_head = pl.pallas_call(
    _head_body,
    out_shape=jax.ShapeDtypeStruct((BATCH, NCLS), jnp.float32),
)


def kernel(x, table, W, b):
    x2 = x.astype(jnp.int32).reshape(BATCH * 2, HALF)
    pooled = _pooled_sum(x2, table)
    return _head(pooled, W, b.reshape(1, NCLS))
